# baseline (speedup 1.0000x reference)
import jax
import jax.numpy as jnp
from jax.experimental import pallas as pl

K = 48
ROWS_PER_STEP = 32
EXACT_ROWS_PER_STEP = 64
GROUP = 8
CHUNKS = 16
LANES = 128
W = 6
BIG_IDX = 4096.0
INF = jnp.inf


def _batcher_pairs(n):
    pairs = []

    def merge(lo, m, r):
        step = r * 2
        if step < m:
            merge(lo, m, step)
            merge(lo + r, m, step)
            for i in range(lo + r, lo + m - r, step):
                pairs.append((i, i + r))
        else:
            pairs.append((lo, lo + r))

    def sort(lo, m):
        if m > 1:
            h = m // 2
            sort(lo, h)
            sort(lo + h, h)
            merge(lo, m, 1)

    sort(0, n)
    return pairs


_PAIRS = _batcher_pairs(CHUNKS)


def _column_sort(x):
    lane_f = jax.lax.broadcasted_iota(jnp.int32, (GROUP, LANES), 1).astype(
        jnp.float32)
    c = [x[:, j * LANES:(j + 1) * LANES] for j in range(CHUNKS)]
    d = [lane_f + float(j * LANES) for j in range(CHUNKS)]
    for a, b in _PAIRS:
        va, vb = c[a], c[b]
        ia, ib = d[a], d[b]
        swap = (vb < va) | ((vb == va) & (ib < ia))
        c[a] = jnp.where(swap, vb, va)
        c[b] = jnp.where(swap, va, vb)
        d[a] = jnp.where(swap, ib, ia)
        d[b] = jnp.where(swap, ia, ib)
    return c, d


def _extract_iter(c, d, depth):
    m = jnp.min(c[0], axis=1, keepdims=True)
    w = jnp.min(jnp.where(c[0] == m, d[0], BIG_IDX), axis=1, keepdims=True)
    mask = d[0] == w
    for j in range(depth - 1):
        c[j] = jnp.where(mask, c[j + 1], c[j])
        d[j] = jnp.where(mask, d[j + 1], d[j])
    c[depth - 1] = jnp.where(mask, INF, c[depth - 1])
    d[depth - 1] = jnp.where(mask, BIG_IDX, d[depth - 1])
    return m, w


def _extract(c, d, depth):
    vals, idxs = [], []
    for _ in range(K):
        m, w = _extract_iter(c, d, depth)
        vals.append(m)
        idxs.append(w)
    return jnp.concatenate(vals, axis=1), jnp.concatenate(idxs, axis=1)


def _fast_kernel(x_ref, idx_ref, val_ref, flag_ref):
    ngroups = ROWS_PER_STEP // GROUP
    cs, ds, guards = [], [], []
    for g in range(ngroups):
        sl = slice(g * GROUP, (g + 1) * GROUP)
        c, d = _column_sort(x_ref[sl, :])
        guards.append(c[W])
        cs.append(c[:W])
        ds.append(d[:W])
    vals = [[] for _ in range(ngroups)]
    idxs = [[] for _ in range(ngroups)]
    for _ in range(K):
        for g in range(ngroups):
            m, w = _extract_iter(cs[g], ds[g], W)
            vals[g].append(m)
            idxs[g].append(w)
    for g in range(ngroups):
        sl = slice(g * GROUP, (g + 1) * GROUP)
        v = jnp.concatenate(vals[g], axis=1)
        w = jnp.concatenate(idxs[g], axis=1)
        v47 = v[:, K - 1:K]
        viol = (cs[g][0] == INF) & (guards[g] <= v47)
        flag_ref[sl, :] = jnp.max(viol.astype(jnp.int32), axis=1, keepdims=True)
        val_ref[sl, :] = v
        idx_ref[sl, :] = w.astype(jnp.int32)


def _exact_kernel(x_ref, idx_ref, val_ref):
    for g in range(EXACT_ROWS_PER_STEP // GROUP):
        sl = slice(g * GROUP, (g + 1) * GROUP)
        c, d = _column_sort(x_ref[sl, :])
        v, w = _extract(c, d, CHUNKS)
        val_ref[sl, :] = v
        idx_ref[sl, :] = w.astype(jnp.int32)


def kernel(D):
    b, q, n = D.shape
    rows = b * q
    Df = D.reshape(rows, n)
    grid = (rows // ROWS_PER_STEP,)
    row_spec = pl.BlockSpec((ROWS_PER_STEP, n), lambda i: (i, 0))
    out_spec = pl.BlockSpec((ROWS_PER_STEP, K), lambda i: (i, 0))
    flag_spec = pl.BlockSpec((ROWS_PER_STEP, 1), lambda i: (i, 0))
    idx, vals, flags = pl.pallas_call(
        _fast_kernel,
        grid=grid,
        in_specs=[row_spec],
        out_specs=[out_spec, out_spec, flag_spec],
        out_shape=[
            jax.ShapeDtypeStruct((rows, K), jnp.int32),
            jax.ShapeDtypeStruct((rows, K), jnp.float32),
            jax.ShapeDtypeStruct((rows, 1), jnp.int32),
        ],
    )(Df)

    def exact(_):
        return pl.pallas_call(
            _exact_kernel,
            grid=(rows // EXACT_ROWS_PER_STEP,),
            in_specs=[pl.BlockSpec((EXACT_ROWS_PER_STEP, n), lambda i: (i, 0))],
            out_specs=[
                pl.BlockSpec((EXACT_ROWS_PER_STEP, K), lambda i: (i, 0)),
                pl.BlockSpec((EXACT_ROWS_PER_STEP, K), lambda i: (i, 0)),
            ],
            out_shape=[
                jax.ShapeDtypeStruct((rows, K), jnp.int32),
                jax.ShapeDtypeStruct((rows, K), jnp.float32),
            ],
        )(Df)

    idx, vals = jax.lax.cond(jnp.max(flags) > 0, exact,
                             lambda _: (idx, vals), None)
    return idx.reshape(b, q, K), vals.reshape(b, q, K)


# speedup vs baseline: 5.1864x; 5.1864x over previous
import jax
import jax.numpy as jnp
from jax.experimental import pallas as pl

K = 48
ROWS_PER_STEP = 256
EXACT_ROWS_PER_STEP = 64
GROUP = 8
CHUNKS = 16
LANES = 128
W = 6
BIG_IDX = 4096.0
INF = jnp.inf


def _batcher_pairs(n):
    pairs = []

    def merge(lo, m, r):
        step = r * 2
        if step < m:
            merge(lo, m, step)
            merge(lo + r, m, step)
            for i in range(lo + r, lo + m - r, step):
                pairs.append((i, i + r))
        else:
            pairs.append((lo, lo + r))

    def sort(lo, m):
        if m > 1:
            h = m // 2
            sort(lo, h)
            sort(lo + h, h)
            merge(lo, m, 1)

    sort(0, n)
    return pairs


_PAIRS = _batcher_pairs(CHUNKS)


def _select_pairs(pairs, nout):
    need = set(range(nout))
    kept = []
    for a, b in reversed(pairs):
        if a in need or b in need:
            kept.append((a, b))
            need.add(a)
            need.add(b)
    kept.reverse()
    return kept


_SEL_PAIRS = _select_pairs(_PAIRS, W + 1)


def _column_sort(x, pairs=_PAIRS):
    lane_f = jax.lax.broadcasted_iota(jnp.int32, (GROUP, LANES), 1).astype(
        jnp.float32)
    c = [x[:, j * LANES:(j + 1) * LANES] for j in range(CHUNKS)]
    d = [lane_f + float(j * LANES) for j in range(CHUNKS)]
    for a, b in pairs:
        va, vb = c[a], c[b]
        ia, ib = d[a], d[b]
        swap = (vb < va) | ((vb == va) & (ib < ia))
        c[a] = jnp.where(swap, vb, va)
        c[b] = jnp.where(swap, va, vb)
        d[a] = jnp.where(swap, ib, ia)
        d[b] = jnp.where(swap, ia, ib)
    return c, d


def _extract_iter(c, d, depth):
    m = jnp.min(c[0], axis=1, keepdims=True)
    w = jnp.min(jnp.where(c[0] == m, d[0], BIG_IDX), axis=1, keepdims=True)
    mask = d[0] == w
    for j in range(depth - 1):
        c[j] = jnp.where(mask, c[j + 1], c[j])
        d[j] = jnp.where(mask, d[j + 1], d[j])
    c[depth - 1] = jnp.where(mask, INF, c[depth - 1])
    d[depth - 1] = jnp.where(mask, BIG_IDX, d[depth - 1])
    return m, w


def _extract(c, d, depth):
    vals, idxs = [], []
    for _ in range(K):
        m, w = _extract_iter(c, d, depth)
        vals.append(m)
        idxs.append(w)
    return jnp.concatenate(vals, axis=1), jnp.concatenate(idxs, axis=1)


def _fast_kernel(x_ref, idx_ref, val_ref, flag_ref):
    ngroups = ROWS_PER_STEP // GROUP
    cs, ds, guards = [], [], []
    for g in range(ngroups):
        sl = slice(g * GROUP, (g + 1) * GROUP)
        c, d = _column_sort(x_ref[sl, :], _SEL_PAIRS)
        guards.append(c[W])
        cs.append(c[:W])
        ds.append(d[:W])
    vals = [[] for _ in range(ngroups)]
    idxs = [[] for _ in range(ngroups)]
    for _ in range(K):
        for g in range(ngroups):
            m, w = _extract_iter(cs[g], ds[g], W)
            vals[g].append(m)
            idxs[g].append(w)
    for g in range(ngroups):
        sl = slice(g * GROUP, (g + 1) * GROUP)
        v = jnp.concatenate(vals[g], axis=1)
        w = jnp.concatenate(idxs[g], axis=1)
        v47 = v[:, K - 1:K]
        viol = (cs[g][0] == INF) & (guards[g] <= v47)
        flag_ref[sl, :] = jnp.max(viol.astype(jnp.int32), axis=1, keepdims=True)
        val_ref[sl, :] = v
        idx_ref[sl, :] = w.astype(jnp.int32)


def _exact_kernel(x_ref, idx_ref, val_ref):
    for g in range(EXACT_ROWS_PER_STEP // GROUP):
        sl = slice(g * GROUP, (g + 1) * GROUP)
        c, d = _column_sort(x_ref[sl, :])
        v, w = _extract(c, d, CHUNKS)
        val_ref[sl, :] = v
        idx_ref[sl, :] = w.astype(jnp.int32)


def kernel(D):
    b, q, n = D.shape
    rows = b * q
    Df = D.reshape(rows, n)
    grid = (rows // ROWS_PER_STEP,)
    row_spec = pl.BlockSpec((ROWS_PER_STEP, n), lambda i: (i, 0))
    out_spec = pl.BlockSpec((ROWS_PER_STEP, K), lambda i: (i, 0))
    flag_spec = pl.BlockSpec((ROWS_PER_STEP, 1), lambda i: (i, 0))
    idx, vals, flags = pl.pallas_call(
        _fast_kernel,
        grid=grid,
        in_specs=[row_spec],
        out_specs=[out_spec, out_spec, flag_spec],
        out_shape=[
            jax.ShapeDtypeStruct((rows, K), jnp.int32),
            jax.ShapeDtypeStruct((rows, K), jnp.float32),
            jax.ShapeDtypeStruct((rows, 1), jnp.int32),
        ],
    )(Df)

    def exact(_):
        return pl.pallas_call(
            _exact_kernel,
            grid=(rows // EXACT_ROWS_PER_STEP,),
            in_specs=[pl.BlockSpec((EXACT_ROWS_PER_STEP, n), lambda i: (i, 0))],
            out_specs=[
                pl.BlockSpec((EXACT_ROWS_PER_STEP, K), lambda i: (i, 0)),
                pl.BlockSpec((EXACT_ROWS_PER_STEP, K), lambda i: (i, 0)),
            ],
            out_shape=[
                jax.ShapeDtypeStruct((rows, K), jnp.int32),
                jax.ShapeDtypeStruct((rows, K), jnp.float32),
            ],
        )(Df)

    idx, vals = jax.lax.cond(jnp.max(flags) > 0, exact,
                             lambda _: (idx, vals), None)
    return idx.reshape(b, q, K), vals.reshape(b, q, K)
